# Initial kernel scaffold; baseline (speedup 1.0000x reference)
import jax
import jax.numpy as jnp
from jax.experimental import pallas as pl


def kernel(x, edge_index, sentence, W1, att_src1, att_dst1, bias1, W2, att_src2, att_dst2, bias2, W_res1, W_res2, W_act, b_act):
    raise NotImplementedError("write your pallas kernel here")


# speedup vs baseline: 1.1812x; 1.1812x over previous
import functools
import jax
import jax.numpy as jnp
from jax.experimental import pallas as pl
from jax.experimental.pallas import tpu as pltpu

N = 10000
E = 320000
F_IN = 128
H1 = 8
HC = 64
R = 1000

_pallas_call = pl.pallas_call


def _prep1_body(x_ref, w1t_ref, wrt_ref, asrc_ref, adst_ref,
                xp_ref, xr_ref, as_ref, ad_ref):
    xp = jnp.dot(x_ref[...], w1t_ref[...], preferred_element_type=jnp.float32)
    xp_ref[...] = xp
    xr_ref[...] = jnp.dot(x_ref[...], wrt_ref[...],
                          preferred_element_type=jnp.float32)
    xph = xp.reshape(R, H1, HC)
    as_ref[...] = (xph * asrc_ref[...]).sum(-1)
    ad_ref[...] = (xph * adst_ref[...]).sum(-1)


def _prep1(x, W1, W_res1, att_src1, att_dst1):
    grid = (N // R,)
    return _pallas_call(
        _prep1_body,
        grid=grid,
        in_specs=[
            pl.BlockSpec((R, F_IN), lambda i: (i, 0)),
            pl.BlockSpec((F_IN, H1 * HC), lambda i: (0, 0)),
            pl.BlockSpec((F_IN, H1 * HC), lambda i: (0, 0)),
            pl.BlockSpec((H1, HC), lambda i: (0, 0)),
            pl.BlockSpec((H1, HC), lambda i: (0, 0)),
        ],
        out_specs=[
            pl.BlockSpec((R, H1 * HC), lambda i: (i, 0)),
            pl.BlockSpec((R, H1 * HC), lambda i: (i, 0)),
            pl.BlockSpec((R, H1), lambda i: (i, 0)),
            pl.BlockSpec((R, H1), lambda i: (i, 0)),
        ],
        out_shape=[
            jax.ShapeDtypeStruct((N, H1 * HC), jnp.float32),
            jax.ShapeDtypeStruct((N, H1 * HC), jnp.float32),
            jax.ShapeDtypeStruct((N, H1), jnp.float32),
            jax.ShapeDtypeStruct((N, H1), jnp.float32),
        ],
    )(x, W1.T, W_res1.T, att_src1[0], att_dst1[0])


def kernel(x, edge_index, sentence, W1, att_src1, att_dst1, bias1,
           W2, att_src2, att_dst2, bias2, W_res1, W_res2, W_act, b_act):
    ei = edge_index.astype(jnp.int32)
    src, dst = ei[0], ei[1]

    xp1, xr1, as1, ad1 = _prep1(x, W1, W_res1, att_src1, att_dst1)

    e1 = as1[src] + ad1[dst]
    e1 = jnp.where(e1 > 0, e1, 0.2 * e1)
    w1 = jnp.exp(e1)
    es = as1 + ad1
    ws = jnp.exp(jnp.where(es > 0, es, 0.2 * es))
    den1 = jax.ops.segment_sum(w1, dst, num_segments=N) + ws
    xph = xp1.reshape(N, H1, HC)
    agg1 = jax.ops.segment_sum(w1[:, :, None] * xph[src], dst,
                               num_segments=N) + ws[:, :, None] * xph

    h1 = (agg1 / (den1[:, :, None] + 1e-16)).reshape(N, H1 * HC) + bias1
    h1 = jax.nn.elu(h1)
    h = h1 + xr1

    xp2, hr2 = _dense2(h, W2, W_res2)
    as2 = (xp2 * att_src2[0, 0][None, :]).sum(-1)
    ad2 = (xp2 * att_dst2[0, 0][None, :]).sum(-1)

    e2 = as2[src] + ad2[dst]
    e2 = jnp.where(e2 > 0, e2, 0.2 * e2)
    w2 = jnp.exp(e2)
    es2 = as2 + ad2
    ws2 = jnp.exp(jnp.where(es2 > 0, es2, 0.2 * es2))
    den2 = jax.ops.segment_sum(w2, dst, num_segments=N) + ws2
    agg2 = jax.ops.segment_sum(w2[:, None] * xp2[src], dst,
                               num_segments=N) + ws2[:, None] * xp2

    h2 = agg2 / (den2[:, None] + 1e-16) + bias2
    h_out = h2 + hr2
    nrm = jnp.sqrt((h_out * h_out).sum(-1, keepdims=True))
    h_out = h_out / jnp.maximum(nrm, 1e-12)
    logits = (h_out @ W_act.T + b_act).reshape(1, -1)
    action = jnp.argmax(logits, axis=1)

    alpha_e = w2 / (den2[dst] + 1e-16)
    alpha_s = ws2 / (den2 + 1e-16)
    alpha2 = jnp.concatenate([alpha_e, alpha_s])[:, None]
    return action, h_out, alpha2, logits


def _dense2_body(h_ref, w2t_ref, wr2t_ref, xp2_ref, hr2_ref):
    h = h_ref[...]
    xp2_ref[...] = jnp.dot(h, w2t_ref[...], preferred_element_type=jnp.float32)
    hr2_ref[...] = jnp.dot(h, wr2t_ref[...], preferred_element_type=jnp.float32)


def _dense2(h, W2, W_res2):
    return _pallas_call(
        _dense2_body,
        grid=(N // R,),
        in_specs=[
            pl.BlockSpec((R, H1 * HC), lambda i: (i, 0)),
            pl.BlockSpec((H1 * HC, HC), lambda i: (0, 0)),
            pl.BlockSpec((H1 * HC, HC), lambda i: (0, 0)),
        ],
        out_specs=[
            pl.BlockSpec((R, HC), lambda i: (i, 0)),
            pl.BlockSpec((R, HC), lambda i: (i, 0)),
        ],
        out_shape=[
            jax.ShapeDtypeStruct((N, HC), jnp.float32),
            jax.ShapeDtypeStruct((N, HC), jnp.float32),
        ],
    )(h, W2.T, W_res2.T)


# speedup vs baseline: 13.9045x; 11.7719x over previous
import functools
import jax
import jax.numpy as jnp
from jax import lax
from jax.experimental import pallas as pl
from jax.experimental.pallas import tpu as pltpu
from jax.experimental.pallas import tpu_sc as plsc

N = 10000
E = 320000
F_IN = 128
H1 = 8
HC = 64
R = 1000
TILES = 32
EPT = E // TILES
C = 80
NCH = EPT // C
ROWS = N // 16
FP = 128

_pallas_call = pl.pallas_call


def _prep1_body(x_ref, w1t_ref, wrt_ref, asrc_ref, adst_ref,
                xp_ref, xr_ref, as_ref, ad_ref):
    xp = jnp.dot(x_ref[...], w1t_ref[...], preferred_element_type=jnp.float32)
    xp_ref[...] = xp
    xr_ref[...] = jnp.dot(x_ref[...], wrt_ref[...],
                          preferred_element_type=jnp.float32)
    xph = xp.reshape(R, H1, HC)
    as_ref[...] = (xph * asrc_ref[...]).sum(-1)
    ad_ref[...] = (xph * adst_ref[...]).sum(-1)


def _prep1(x, W1, W_res1, att_src1, att_dst1):
    return _pallas_call(
        _prep1_body,
        grid=(N // R,),
        in_specs=[
            pl.BlockSpec((R, F_IN), lambda i: (i, 0)),
            pl.BlockSpec((F_IN, H1 * HC), lambda i: (0, 0)),
            pl.BlockSpec((F_IN, H1 * HC), lambda i: (0, 0)),
            pl.BlockSpec((H1, HC), lambda i: (0, 0)),
            pl.BlockSpec((H1, HC), lambda i: (0, 0)),
        ],
        out_specs=[
            pl.BlockSpec((R, H1 * HC), lambda i: (i, 0)),
            pl.BlockSpec((R, H1 * HC), lambda i: (i, 0)),
            pl.BlockSpec((R, H1), lambda i: (i, 0)),
            pl.BlockSpec((R, H1), lambda i: (i, 0)),
        ],
        out_shape=[
            jax.ShapeDtypeStruct((N, H1 * HC), jnp.float32),
            jax.ShapeDtypeStruct((N, H1 * HC), jnp.float32),
            jax.ShapeDtypeStruct((N, H1), jnp.float32),
            jax.ShapeDtypeStruct((N, H1), jnp.float32),
        ],
    )(x, W1.T, W_res1.T, att_src1[0], att_dst1[0])


def _mid_body(agg_ref, den_ref, xp_ref, as_ref, ad_ref, xr_ref, b1_ref,
              w2t_ref, wr2t_ref, xp2_ref, hr2_ref):
    es = as_ref[...] + ad_ref[...]
    ws = jnp.exp(jnp.maximum(es, 0.2 * es))
    xph = xp_ref[...].reshape(R, H1, HC)
    agg = agg_ref[...].reshape(R, H1, HC) + ws[:, :, None] * xph
    den = den_ref[...] + ws
    h1 = (agg / (den[:, :, None] + 1e-16)).reshape(R, H1 * HC) + b1_ref[...]
    h1 = jnp.where(h1 > 0, h1, jnp.exp(h1) - 1.0)
    h = h1 + xr_ref[...]
    xp2_ref[...] = jnp.dot(h, w2t_ref[...], preferred_element_type=jnp.float32)
    hr2_ref[...] = jnp.dot(h, wr2t_ref[...],
                           preferred_element_type=jnp.float32)


def _mid(agg1, den1, xp1, as1, ad1, xr1, bias1, W2, W_res2):
    return _pallas_call(
        _mid_body,
        grid=(N // R,),
        in_specs=[
            pl.BlockSpec((R, H1 * HC), lambda i: (i, 0)),
            pl.BlockSpec((R, H1), lambda i: (i, 0)),
            pl.BlockSpec((R, H1 * HC), lambda i: (i, 0)),
            pl.BlockSpec((R, H1), lambda i: (i, 0)),
            pl.BlockSpec((R, H1), lambda i: (i, 0)),
            pl.BlockSpec((R, H1 * HC), lambda i: (i, 0)),
            pl.BlockSpec((1, H1 * HC), lambda i: (0, 0)),
            pl.BlockSpec((H1 * HC, HC), lambda i: (0, 0)),
            pl.BlockSpec((H1 * HC, HC), lambda i: (0, 0)),
        ],
        out_specs=[
            pl.BlockSpec((R, HC), lambda i: (i, 0)),
            pl.BlockSpec((R, HC), lambda i: (i, 0)),
        ],
        out_shape=[
            jax.ShapeDtypeStruct((N, HC), jnp.float32),
            jax.ShapeDtypeStruct((N, HC), jnp.float32),
        ],
    )(agg1, den1, xp1, as1, ad1, xr1, bias1.reshape(1, -1), W2.T, W_res2.T)


def _fin_body(agg_ref, den_ref, xp2_ref, es_ref, hr2_ref, b2_ref, wact_ref,
              hout_ref, logit_ref):
    ws2 = jnp.exp(jnp.maximum(es_ref[...], 0.2 * es_ref[...]))
    agg = agg_ref[...] + ws2 * xp2_ref[...]
    den = den_ref[...] + ws2
    h2 = agg / (den + 1e-16) + b2_ref[...]
    h_out = h2 + hr2_ref[...]
    nrm = jnp.sqrt((h_out * h_out).sum(-1, keepdims=True))
    h_out = h_out / jnp.maximum(nrm, 1e-12)
    hout_ref[...] = h_out
    logit_ref[...] = (h_out * wact_ref[...]).sum(-1, keepdims=True)


def _fin(agg2, den2, xp2, es2, hr2, bias2, W_act):
    return _pallas_call(
        _fin_body,
        grid=(N // R,),
        in_specs=[
            pl.BlockSpec((R, HC), lambda i: (i, 0)),
            pl.BlockSpec((R, 1), lambda i: (i, 0)),
            pl.BlockSpec((R, HC), lambda i: (i, 0)),
            pl.BlockSpec((R, 1), lambda i: (i, 0)),
            pl.BlockSpec((R, HC), lambda i: (i, 0)),
            pl.BlockSpec((1, HC), lambda i: (0, 0)),
            pl.BlockSpec((1, HC), lambda i: (0, 0)),
        ],
        out_specs=[
            pl.BlockSpec((R, HC), lambda i: (i, 0)),
            pl.BlockSpec((R, 1), lambda i: (i, 0)),
        ],
        out_shape=[
            jax.ShapeDtypeStruct((N, HC), jnp.float32),
            jax.ShapeDtypeStruct((N, 1), jnp.float32),
        ],
    )(agg2, den2[:, None], xp2, es2[:, None], hr2, bias2.reshape(1, -1),
      W_act.reshape(1, -1))


_I16 = functools.partial(jax.lax.iota, jnp.int32, 16)


def _edge_weights(asv, adv, sv16, dv16, h):
    a = plsc.load_gather(asv, [sv16 * 2 + h])
    b = plsc.load_gather(adv, [dv16 * 2 + h])
    e = a + b
    return jnp.exp(jnp.maximum(e, 0.2 * e))


def _sc_agg1(xp_tab, src, dst, asp, adp):
    mesh = plsc.VectorSubcoreMesh(core_axis_name="c", subcore_axis_name="s")

    @functools.partial(
        pl.kernel,
        out_type=[
            jax.ShapeDtypeStruct((2, H1, N, HC), jnp.float32),
            jax.ShapeDtypeStruct((2, H1, N, 16), jnp.float32),
        ],
        mesh=mesh,
        compiler_params=pltpu.CompilerParams(needs_layout_passes=False, use_tc_tiling_on_sc=False),
        scratch_types=[
            pltpu.VMEM((C,), jnp.int32),
            pltpu.VMEM((C,), jnp.int32),
            pltpu.VMEM((C,), jnp.int32),
            pltpu.VMEM((C, HC), jnp.float32),
            pltpu.VMEM((C, 16), jnp.float32),
            pltpu.VMEM((N,), jnp.float32),
            pltpu.VMEM((N,), jnp.float32),
            pltpu.VMEM((16, HC), jnp.float32),
            pltpu.VMEM((16, 16), jnp.float32),
            pltpu.VMEM_SHARED((N, HC), jnp.float32),
            pltpu.VMEM_SHARED((N, 16), jnp.float32),
            pltpu.SemaphoreType.DMA,
        ],
    )
    def k(xp_ref, src_ref, dst_ref, asp_ref, adp_ref, feat_out, w_out,
          sv, dv, gv, gbuf, wbuf, asv, adv, zbuf, zwbuf, accf, accw, sem):
        c = lax.axis_index("c")
        s = lax.axis_index("s")
        tile = c * 16 + s
        ebase = tile * EPT
        row0 = s * 624
        nblk = jnp.where(s == 15, 40, 39)

        def zero_row(r, _):
            for q in range(HC // 16):
                zbuf[r, pl.ds(q * 16, 16)] = jnp.zeros((16,), jnp.float32)
            zwbuf[r, pl.ds(0, 16)] = jnp.zeros((16,), jnp.float32)
            return 0

        lax.fori_loop(0, 16, zero_row, 0)

        def zero_wbuf(r, _):
            wbuf[r, pl.ds(0, 16)] = jnp.zeros((16,), jnp.float32)
            return 0

        lax.fori_loop(0, C, zero_wbuf, 0)

        for p in range(H1):
            pltpu.sync_copy(asp_ref.at[pl.ds(p * N, N)], asv)
            pltpu.sync_copy(adp_ref.at[pl.ds(p * N, N)], adv)

            def zero_acc(b, _):
                sl = pl.ds(row0 + b * 16, 16)
                pltpu.sync_copy(zbuf, accf.at[sl])
                pltpu.sync_copy(zwbuf, accw.at[sl])
                return 0

            lax.fori_loop(0, nblk, zero_acc, 0)
            plsc.subcore_barrier()

            def chunk(kk, _):
                base = ebase + kk * C
                pltpu.sync_copy(src_ref.at[pl.ds(base, C)], sv)
                pltpu.sync_copy(dst_ref.at[pl.ds(base, C)], dv)
                wvecs = []
                for g in range(C // 16):
                    sv16 = sv[pl.ds(g * 16, 16)]
                    dv16 = dv[pl.ds(g * 16, 16)]
                    gv[pl.ds(g * 16, 16)] = sv16 * H1 + p
                    a = plsc.load_gather(asv, [sv16])
                    b = plsc.load_gather(adv, [dv16])
                    e = a + b
                    w = jnp.exp(jnp.maximum(e, 0.2 * e))
                    rows = _I16() + g * 16
                    plsc.store_scatter(wbuf, [rows, jnp.zeros((16,), jnp.int32)], w)
                    wvecs.append(w)
                pltpu.async_copy(xp_ref.at[gv], gbuf, sem).wait()
                for g in range(C // 16):
                    w = wvecs[g]
                    for j in range(16):
                        r = g * 16 + j
                        wj = w[j]
                        for q in range(HC // 16):
                            gbuf[r, pl.ds(q * 16, 16)] = gbuf[r, pl.ds(q * 16, 16)] * wj
                pltpu.sync_copy(gbuf, accf.at[dv], add=True)
                pltpu.sync_copy(wbuf, accw.at[dv], add=True)
                return 0

            lax.fori_loop(0, NCH, chunk, 0)
            plsc.subcore_barrier()

            def dump(b, _):
                sl = pl.ds(row0 + b * 16, 16)
                pltpu.sync_copy(accf.at[sl], feat_out.at[c, p, sl])
                pltpu.sync_copy(accw.at[sl], w_out.at[c, p, sl])
                return 0

            lax.fori_loop(0, nblk, dump, 0)
            plsc.subcore_barrier()

    return k(xp_tab, src, dst, asp, adp)


def _sc_agg2(xp2, src, dst, as2, ad2):
    mesh = plsc.VectorSubcoreMesh(core_axis_name="c", subcore_axis_name="s")

    @functools.partial(
        pl.kernel,
        out_type=[
            jax.ShapeDtypeStruct((2, N, HC), jnp.float32),
            jax.ShapeDtypeStruct((2, N, 16), jnp.float32),
            jax.ShapeDtypeStruct((E,), jnp.float32),
        ],
        mesh=mesh,
        compiler_params=pltpu.CompilerParams(needs_layout_passes=False, use_tc_tiling_on_sc=False),
        scratch_types=[
            pltpu.VMEM((C,), jnp.int32),
            pltpu.VMEM((C,), jnp.int32),
            pltpu.VMEM((C, HC), jnp.float32),
            pltpu.VMEM((C, 16), jnp.float32),
            pltpu.VMEM((C,), jnp.float32),
            pltpu.VMEM((N,), jnp.float32),
            pltpu.VMEM((N,), jnp.float32),
            pltpu.VMEM((16, HC), jnp.float32),
            pltpu.VMEM((16, 16), jnp.float32),
            pltpu.VMEM_SHARED((N, HC), jnp.float32),
            pltpu.VMEM_SHARED((N, 16), jnp.float32),
            pltpu.SemaphoreType.DMA,
        ],
    )
    def k(xp_ref, src_ref, dst_ref, as_ref, ad_ref, feat_out, w_out, wraw_out,
          sv, dv, gbuf, wbuf, wlin, asv, adv, zbuf, zwbuf, accf, accw, sem):
        c = lax.axis_index("c")
        s = lax.axis_index("s")
        tile = c * 16 + s
        ebase = tile * EPT
        row0 = s * 624
        nblk = jnp.where(s == 15, 40, 39)

        def zero_row(r, _):
            for q in range(HC // 16):
                zbuf[r, pl.ds(q * 16, 16)] = jnp.zeros((16,), jnp.float32)
            zwbuf[r, pl.ds(0, 16)] = jnp.zeros((16,), jnp.float32)
            return 0

        lax.fori_loop(0, 16, zero_row, 0)

        def zero_wbuf(r, _):
            wbuf[r, pl.ds(0, 16)] = jnp.zeros((16,), jnp.float32)
            return 0

        lax.fori_loop(0, C, zero_wbuf, 0)

        pltpu.sync_copy(as_ref, asv)
        pltpu.sync_copy(ad_ref, adv)

        def zero_acc(b, _):
            sl = pl.ds(row0 + b * 16, 16)
            pltpu.sync_copy(zbuf, accf.at[sl])
            pltpu.sync_copy(zwbuf, accw.at[sl])
            return 0

        lax.fori_loop(0, nblk, zero_acc, 0)
        plsc.subcore_barrier()

        def chunk(kk, _):
            base = ebase + kk * C
            pltpu.sync_copy(src_ref.at[pl.ds(base, C)], sv)
            pltpu.sync_copy(dst_ref.at[pl.ds(base, C)], dv)
            wvecs = []
            for g in range(C // 16):
                sv16 = sv[pl.ds(g * 16, 16)]
                dv16 = dv[pl.ds(g * 16, 16)]
                a = plsc.load_gather(asv, [sv16])
                bb = plsc.load_gather(adv, [dv16])
                e = a + bb
                w = jnp.exp(jnp.maximum(e, 0.2 * e))
                rows = _I16() + g * 16
                plsc.store_scatter(wbuf, [rows, jnp.zeros((16,), jnp.int32)], w)
                wlin[pl.ds(g * 16, 16)] = w
                wvecs.append(w)
            pltpu.async_copy(xp_ref.at[sv], gbuf, sem).wait()
            for g in range(C // 16):
                w = wvecs[g]
                for j in range(16):
                    r = g * 16 + j
                    wj = w[j]
                    for q in range(4):
                        gbuf[r, pl.ds(q * 16, 16)] = gbuf[r, pl.ds(q * 16, 16)] * wj
            pltpu.sync_copy(gbuf, accf.at[dv], add=True)
            pltpu.sync_copy(wbuf, accw.at[dv], add=True)
            pltpu.sync_copy(wlin, wraw_out.at[pl.ds(base, C)])
            return 0

        lax.fori_loop(0, NCH, chunk, 0)
        plsc.subcore_barrier()

        def dump(b, _):
            sl = pl.ds(row0 + b * 16, 16)
            pltpu.sync_copy(accf.at[sl], feat_out.at[c, sl])
            pltpu.sync_copy(accw.at[sl], w_out.at[c, sl])
            return 0

        lax.fori_loop(0, nblk, dump, 0)
        plsc.subcore_barrier()

    return k(xp2, src, dst, as2, ad2)


def _sc_alpha(w2, dst, den2):
    mesh = plsc.VectorSubcoreMesh(core_axis_name="c", subcore_axis_name="s")

    @functools.partial(
        pl.kernel,
        out_type=jax.ShapeDtypeStruct((E,), jnp.float32),
        mesh=mesh,
        compiler_params=pltpu.CompilerParams(needs_layout_passes=False, use_tc_tiling_on_sc=False),
        scratch_types=[
            pltpu.VMEM((C,), jnp.int32),
            pltpu.VMEM((C,), jnp.float32),
            pltpu.VMEM((C,), jnp.float32),
            pltpu.VMEM((N,), jnp.float32),
        ],
    )
    def k(w_ref, dst_ref, den_ref, a_out, dv, wv, av, denv):
        c = lax.axis_index("c")
        s = lax.axis_index("s")
        tile = c * 16 + s
        ebase = tile * EPT
        pltpu.sync_copy(den_ref, denv)

        def chunk(kk, _):
            base = ebase + kk * C
            pltpu.sync_copy(dst_ref.at[pl.ds(base, C)], dv)
            pltpu.sync_copy(w_ref.at[pl.ds(base, C)], wv)
            for g in range(C // 16):
                dv16 = dv[pl.ds(g * 16, 16)]
                d = plsc.load_gather(denv, [dv16])
                av[pl.ds(g * 16, 16)] = wv[pl.ds(g * 16, 16)] / (d + 1e-16)
            pltpu.sync_copy(av, a_out.at[pl.ds(base, C)])
            return 0

        lax.fori_loop(0, NCH, chunk, 0)

    return k(w2, dst, den2)


def kernel(x, edge_index, sentence, W1, att_src1, att_dst1, bias1,
           W2, att_src2, att_dst2, bias2, W_res1, W_res2, W_act, b_act):
    ei = edge_index.astype(jnp.int32)
    src, dst = ei[0], ei[1]

    xp1, xr1, as1, ad1 = _prep1(x, W1, W_res1, att_src1, att_dst1)

    asp = as1.T.reshape(H1 * N)
    adp = ad1.T.reshape(H1 * N)
    xp_tab = xp1.reshape(N * H1, HC)
    featp, wp = _sc_agg1(xp_tab, src, dst, asp, adp)
    feat = featp[0] + featp[1]
    agg1 = feat.transpose(1, 0, 2).reshape(N, H1 * HC)
    wsum = wp[0] + wp[1]
    den1 = wsum[:, :, 0].T

    xp2, hr2 = _mid(agg1, den1, xp1, as1, ad1, xr1, bias1, W2, W_res2)
    as2 = (xp2 * att_src2[0, 0][None, :]).sum(-1)
    ad2 = (xp2 * att_dst2[0, 0][None, :]).sum(-1)

    featp2, wp2, w2 = _sc_agg2(xp2, src, dst, as2, ad2)
    agg2 = featp2[0] + featp2[1]
    den2p = wp2[0, :, 0] + wp2[1, :, 0]

    es2 = as2 + ad2
    ws2 = jnp.exp(jnp.maximum(es2, 0.2 * es2))
    den2 = den2p + ws2

    h_out, logits_col = _fin(agg2, den2p, xp2, es2, hr2, bias2, W_act)
    logits = logits_col.reshape(1, -1) + b_act
    action = jnp.argmax(logits, axis=1)

    alpha_e = _sc_alpha(w2, dst, den2)
    alpha_s = ws2 / (den2 + 1e-16)
    alpha2 = jnp.concatenate([alpha_e, alpha_s])[:, None]
    return action, h_out, alpha2, logits


# speedup vs baseline: 18.9474x; 1.3627x over previous
import functools
import jax
import jax.numpy as jnp
from jax import lax
from jax.experimental import pallas as pl
from jax.experimental.pallas import tpu as pltpu
from jax.experimental.pallas import tpu_sc as plsc

N = 10000
E = 320000
F_IN = 128
H1 = 8
HC = 64
R = 1000
TILES = 32
EPT = E // TILES
C = 80
NCH = EPT // C
ROWS = N // 16
FP = 128

_pallas_call = pl.pallas_call


def _prep1_body(x_ref, w1t_ref, wrt_ref, asrc_ref, adst_ref,
                xp_ref, xr_ref, as_ref, ad_ref):
    xp = jnp.dot(x_ref[...], w1t_ref[...], preferred_element_type=jnp.float32)
    xp_ref[...] = xp
    xr_ref[...] = jnp.dot(x_ref[...], wrt_ref[...],
                          preferred_element_type=jnp.float32)
    xph = xp.reshape(R, H1, HC)
    as_ref[...] = (xph * asrc_ref[...]).sum(-1)
    ad_ref[...] = (xph * adst_ref[...]).sum(-1)


def _prep1(x, W1, W_res1, att_src1, att_dst1):
    return _pallas_call(
        _prep1_body,
        grid=(N // R,),
        in_specs=[
            pl.BlockSpec((R, F_IN), lambda i: (i, 0)),
            pl.BlockSpec((F_IN, H1 * HC), lambda i: (0, 0)),
            pl.BlockSpec((F_IN, H1 * HC), lambda i: (0, 0)),
            pl.BlockSpec((H1, HC), lambda i: (0, 0)),
            pl.BlockSpec((H1, HC), lambda i: (0, 0)),
        ],
        out_specs=[
            pl.BlockSpec((R, H1 * HC), lambda i: (i, 0)),
            pl.BlockSpec((R, H1 * HC), lambda i: (i, 0)),
            pl.BlockSpec((R, H1), lambda i: (i, 0)),
            pl.BlockSpec((R, H1), lambda i: (i, 0)),
        ],
        out_shape=[
            jax.ShapeDtypeStruct((N, H1 * HC), jnp.float32),
            jax.ShapeDtypeStruct((N, H1 * HC), jnp.float32),
            jax.ShapeDtypeStruct((N, H1), jnp.float32),
            jax.ShapeDtypeStruct((N, H1), jnp.float32),
        ],
    )(x, W1.T, W_res1.T, att_src1[0], att_dst1[0])


def _mid_body(agg_ref, den_ref, xp_ref, as_ref, ad_ref, xr_ref, b1_ref,
              w2t_ref, wr2t_ref, xp2_ref, hr2_ref):
    es = as_ref[...] + ad_ref[...]
    ws = jnp.exp(jnp.maximum(es, 0.2 * es))
    xph = xp_ref[...].reshape(R, H1, HC)
    agg = agg_ref[...].reshape(R, H1, HC) + ws[:, :, None] * xph
    den = den_ref[...] + ws
    h1 = (agg / (den[:, :, None] + 1e-16)).reshape(R, H1 * HC) + b1_ref[...]
    h1 = jnp.where(h1 > 0, h1, jnp.exp(h1) - 1.0)
    h = h1 + xr_ref[...]
    xp2_ref[...] = jnp.dot(h, w2t_ref[...], preferred_element_type=jnp.float32)
    hr2_ref[...] = jnp.dot(h, wr2t_ref[...],
                           preferred_element_type=jnp.float32)


def _mid(agg1, den1, xp1, as1, ad1, xr1, bias1, W2, W_res2):
    return _pallas_call(
        _mid_body,
        grid=(N // R,),
        in_specs=[
            pl.BlockSpec((R, H1 * HC), lambda i: (i, 0)),
            pl.BlockSpec((R, H1), lambda i: (i, 0)),
            pl.BlockSpec((R, H1 * HC), lambda i: (i, 0)),
            pl.BlockSpec((R, H1), lambda i: (i, 0)),
            pl.BlockSpec((R, H1), lambda i: (i, 0)),
            pl.BlockSpec((R, H1 * HC), lambda i: (i, 0)),
            pl.BlockSpec((1, H1 * HC), lambda i: (0, 0)),
            pl.BlockSpec((H1 * HC, HC), lambda i: (0, 0)),
            pl.BlockSpec((H1 * HC, HC), lambda i: (0, 0)),
        ],
        out_specs=[
            pl.BlockSpec((R, HC), lambda i: (i, 0)),
            pl.BlockSpec((R, HC), lambda i: (i, 0)),
        ],
        out_shape=[
            jax.ShapeDtypeStruct((N, HC), jnp.float32),
            jax.ShapeDtypeStruct((N, HC), jnp.float32),
        ],
    )(agg1, den1, xp1, as1, ad1, xr1, bias1.reshape(1, -1), W2.T, W_res2.T)


def _fin_body(agg_ref, den_ref, xp2_ref, es_ref, hr2_ref, b2_ref, wact_ref,
              hout_ref, logit_ref):
    ws2 = jnp.exp(jnp.maximum(es_ref[...], 0.2 * es_ref[...]))
    agg = agg_ref[...] + ws2 * xp2_ref[...]
    den = den_ref[...] + ws2
    h2 = agg / (den + 1e-16) + b2_ref[...]
    h_out = h2 + hr2_ref[...]
    nrm = jnp.sqrt((h_out * h_out).sum(-1, keepdims=True))
    h_out = h_out / jnp.maximum(nrm, 1e-12)
    hout_ref[...] = h_out
    logit_ref[...] = (h_out * wact_ref[...]).sum(-1, keepdims=True)


def _fin(agg2, den2, xp2, es2, hr2, bias2, W_act):
    return _pallas_call(
        _fin_body,
        grid=(N // R,),
        in_specs=[
            pl.BlockSpec((R, HC), lambda i: (i, 0)),
            pl.BlockSpec((R, 1), lambda i: (i, 0)),
            pl.BlockSpec((R, HC), lambda i: (i, 0)),
            pl.BlockSpec((R, 1), lambda i: (i, 0)),
            pl.BlockSpec((R, HC), lambda i: (i, 0)),
            pl.BlockSpec((1, HC), lambda i: (0, 0)),
            pl.BlockSpec((1, HC), lambda i: (0, 0)),
        ],
        out_specs=[
            pl.BlockSpec((R, HC), lambda i: (i, 0)),
            pl.BlockSpec((R, 1), lambda i: (i, 0)),
        ],
        out_shape=[
            jax.ShapeDtypeStruct((N, HC), jnp.float32),
            jax.ShapeDtypeStruct((N, 1), jnp.float32),
        ],
    )(agg2, den2[:, None], xp2, es2[:, None], hr2, bias2.reshape(1, -1),
      W_act.reshape(1, -1))


_I16 = functools.partial(jax.lax.iota, jnp.int32, 16)


def _edge_weights(asv, adv, sv16, dv16, h):
    a = plsc.load_gather(asv, [sv16 * 2 + h])
    b = plsc.load_gather(adv, [dv16 * 2 + h])
    e = a + b
    return jnp.exp(jnp.maximum(e, 0.2 * e))


def _sc_agg1(xp_tab, src, dst, asp, adp):
    mesh = plsc.VectorSubcoreMesh(core_axis_name="c", subcore_axis_name="s")

    @functools.partial(
        pl.kernel,
        out_type=[
            jax.ShapeDtypeStruct((2, H1, N, HC), jnp.float32),
            jax.ShapeDtypeStruct((2, H1, N, 16), jnp.float32),
        ],
        mesh=mesh,
        compiler_params=pltpu.CompilerParams(needs_layout_passes=False, use_tc_tiling_on_sc=False),
        scratch_types=[
            pltpu.VMEM((EPT,), jnp.int32),
            pltpu.VMEM((C,), jnp.int32),
            pltpu.VMEM((EPT,), jnp.int32),
            pltpu.VMEM((C, HC), jnp.float32),
            pltpu.VMEM((C, 16), jnp.float32),
            pltpu.VMEM((N,), jnp.float32),
            pltpu.VMEM((N,), jnp.float32),
            pltpu.VMEM((16, HC), jnp.float32),
            pltpu.VMEM((16, 16), jnp.float32),
            pltpu.VMEM_SHARED((N, HC), jnp.float32),
            pltpu.VMEM_SHARED((N, 16), jnp.float32),
            pltpu.SemaphoreType.DMA,
        ],
    )
    def k(xp_ref, src_ref, dst_ref, asp_ref, adp_ref, feat_out, w_out,
          sall, dv, gvall, gbuf, wbuf, asv, adv, zbuf, zwbuf, accf, accw, sem):
        c = lax.axis_index("c")
        s = lax.axis_index("s")
        tile = c * 16 + s
        ebase = tile * EPT
        row0 = s * 624
        nblk = jnp.where(s == 15, 40, 39)

        pltpu.sync_copy(src_ref.at[pl.ds(ebase, EPT)], sall)

        def zero_row(r, _):
            for q in range(HC // 16):
                zbuf[r, pl.ds(q * 16, 16)] = jnp.zeros((16,), jnp.float32)
            zwbuf[r, pl.ds(0, 16)] = jnp.zeros((16,), jnp.float32)
            return 0

        lax.fori_loop(0, 16, zero_row, 0)

        def zero_wbuf(r, _):
            wbuf[r, pl.ds(0, 16)] = jnp.zeros((16,), jnp.float32)
            return 0

        lax.fori_loop(0, C, zero_wbuf, 0)

        for p in range(H1):
            pltpu.sync_copy(asp_ref.at[pl.ds(p * N, N)], asv)
            pltpu.sync_copy(adp_ref.at[pl.ds(p * N, N)], adv)

            def gidx(i, _):
                sl = pl.ds(i * 16, 16)
                gvall[sl] = sall[sl] * H1 + p
                return 0

            lax.fori_loop(0, EPT // 16, gidx, 0)

            def zero_acc(b, _):
                sl = pl.ds(row0 + b * 16, 16)
                pltpu.sync_copy(zbuf, accf.at[sl])
                pltpu.sync_copy(zwbuf, accw.at[sl])
                return 0

            lax.fori_loop(0, nblk, zero_acc, 0)
            plsc.subcore_barrier()

            def chunk(kk, _):
                base = ebase + kk * C
                cpy = pltpu.async_copy(
                    xp_ref.at[gvall.at[pl.ds(kk * C, C)]], gbuf, sem)
                pltpu.sync_copy(dst_ref.at[pl.ds(base, C)], dv)
                wvecs = []
                for g in range(C // 16):
                    sv16 = sall[pl.ds(kk * C + g * 16, 16)]
                    dv16 = dv[pl.ds(g * 16, 16)]
                    a = plsc.load_gather(asv, [sv16])
                    b = plsc.load_gather(adv, [dv16])
                    e = a + b
                    w = jnp.exp(jnp.maximum(e, 0.2 * e))
                    rows = _I16() + g * 16
                    plsc.store_scatter(wbuf, [rows, jnp.zeros((16,), jnp.int32)], w)
                    wvecs.append(w)
                cpy.wait()
                for g in range(C // 16):
                    w = wvecs[g]
                    for j in range(16):
                        r = g * 16 + j
                        wj = w[j]
                        for q in range(HC // 16):
                            gbuf[r, pl.ds(q * 16, 16)] = gbuf[r, pl.ds(q * 16, 16)] * wj
                pltpu.sync_copy(gbuf, accf.at[dv], add=True)
                pltpu.sync_copy(wbuf, accw.at[dv], add=True)
                return 0

            lax.fori_loop(0, NCH, chunk, 0)
            plsc.subcore_barrier()

            def dump(b, _):
                sl = pl.ds(row0 + b * 16, 16)
                pltpu.sync_copy(accf.at[sl], feat_out.at[c, p, sl])
                pltpu.sync_copy(accw.at[sl], w_out.at[c, p, sl])
                return 0

            lax.fori_loop(0, nblk, dump, 0)
            plsc.subcore_barrier()

    return k(xp_tab, src, dst, asp, adp)


def _sc_agg2(xp2, src, dst, as2, ad2):
    mesh = plsc.VectorSubcoreMesh(core_axis_name="c", subcore_axis_name="s")

    @functools.partial(
        pl.kernel,
        out_type=[
            jax.ShapeDtypeStruct((2, N, HC), jnp.float32),
            jax.ShapeDtypeStruct((2, N, 16), jnp.float32),
            jax.ShapeDtypeStruct((E,), jnp.float32),
        ],
        mesh=mesh,
        compiler_params=pltpu.CompilerParams(needs_layout_passes=False, use_tc_tiling_on_sc=False),
        scratch_types=[
            pltpu.VMEM((EPT,), jnp.int32),
            pltpu.VMEM((C,), jnp.int32),
            pltpu.VMEM((C, HC), jnp.float32),
            pltpu.VMEM((C, 16), jnp.float32),
            pltpu.VMEM((C,), jnp.float32),
            pltpu.VMEM((N,), jnp.float32),
            pltpu.VMEM((N,), jnp.float32),
            pltpu.VMEM((16, HC), jnp.float32),
            pltpu.VMEM((16, 16), jnp.float32),
            pltpu.VMEM_SHARED((N, HC), jnp.float32),
            pltpu.VMEM_SHARED((N, 16), jnp.float32),
            pltpu.SemaphoreType.DMA,
        ],
    )
    def k(xp_ref, src_ref, dst_ref, as_ref, ad_ref, feat_out, w_out, wraw_out,
          sall, dv, gbuf, wbuf, wlin, asv, adv, zbuf, zwbuf, accf, accw, sem):
        c = lax.axis_index("c")
        s = lax.axis_index("s")
        tile = c * 16 + s
        ebase = tile * EPT
        row0 = s * 624
        nblk = jnp.where(s == 15, 40, 39)

        def zero_row(r, _):
            for q in range(HC // 16):
                zbuf[r, pl.ds(q * 16, 16)] = jnp.zeros((16,), jnp.float32)
            zwbuf[r, pl.ds(0, 16)] = jnp.zeros((16,), jnp.float32)
            return 0

        lax.fori_loop(0, 16, zero_row, 0)

        def zero_wbuf(r, _):
            wbuf[r, pl.ds(0, 16)] = jnp.zeros((16,), jnp.float32)
            return 0

        lax.fori_loop(0, C, zero_wbuf, 0)

        pltpu.sync_copy(as_ref, asv)
        pltpu.sync_copy(ad_ref, adv)
        pltpu.sync_copy(src_ref.at[pl.ds(ebase, EPT)], sall)

        def zero_acc(b, _):
            sl = pl.ds(row0 + b * 16, 16)
            pltpu.sync_copy(zbuf, accf.at[sl])
            pltpu.sync_copy(zwbuf, accw.at[sl])
            return 0

        lax.fori_loop(0, nblk, zero_acc, 0)
        plsc.subcore_barrier()

        def chunk(kk, _):
            base = ebase + kk * C
            cpy = pltpu.async_copy(
                xp_ref.at[sall.at[pl.ds(kk * C, C)]], gbuf, sem)
            pltpu.sync_copy(dst_ref.at[pl.ds(base, C)], dv)
            wvecs = []
            for g in range(C // 16):
                sv16 = sall[pl.ds(kk * C + g * 16, 16)]
                dv16 = dv[pl.ds(g * 16, 16)]
                a = plsc.load_gather(asv, [sv16])
                bb = plsc.load_gather(adv, [dv16])
                e = a + bb
                w = jnp.exp(jnp.maximum(e, 0.2 * e))
                rows = _I16() + g * 16
                plsc.store_scatter(wbuf, [rows, jnp.zeros((16,), jnp.int32)], w)
                wlin[pl.ds(g * 16, 16)] = w
                wvecs.append(w)
            cpy.wait()
            for g in range(C // 16):
                w = wvecs[g]
                for j in range(16):
                    r = g * 16 + j
                    wj = w[j]
                    for q in range(4):
                        gbuf[r, pl.ds(q * 16, 16)] = gbuf[r, pl.ds(q * 16, 16)] * wj
            pltpu.sync_copy(gbuf, accf.at[dv], add=True)
            pltpu.sync_copy(wbuf, accw.at[dv], add=True)
            pltpu.sync_copy(wlin, wraw_out.at[pl.ds(base, C)])
            return 0

        lax.fori_loop(0, NCH, chunk, 0)
        plsc.subcore_barrier()

        def dump(b, _):
            sl = pl.ds(row0 + b * 16, 16)
            pltpu.sync_copy(accf.at[sl], feat_out.at[c, sl])
            pltpu.sync_copy(accw.at[sl], w_out.at[c, sl])
            return 0

        lax.fori_loop(0, nblk, dump, 0)
        plsc.subcore_barrier()

    return k(xp2, src, dst, as2, ad2)


def _sc_alpha(w2, dst, den2):
    mesh = plsc.VectorSubcoreMesh(core_axis_name="c", subcore_axis_name="s")

    @functools.partial(
        pl.kernel,
        out_type=jax.ShapeDtypeStruct((E,), jnp.float32),
        mesh=mesh,
        compiler_params=pltpu.CompilerParams(needs_layout_passes=False, use_tc_tiling_on_sc=False),
        scratch_types=[
            pltpu.VMEM((C,), jnp.int32),
            pltpu.VMEM((C,), jnp.float32),
            pltpu.VMEM((C,), jnp.float32),
            pltpu.VMEM((N,), jnp.float32),
        ],
    )
    def k(w_ref, dst_ref, den_ref, a_out, dv, wv, av, denv):
        c = lax.axis_index("c")
        s = lax.axis_index("s")
        tile = c * 16 + s
        ebase = tile * EPT
        pltpu.sync_copy(den_ref, denv)

        def chunk(kk, _):
            base = ebase + kk * C
            pltpu.sync_copy(dst_ref.at[pl.ds(base, C)], dv)
            pltpu.sync_copy(w_ref.at[pl.ds(base, C)], wv)
            for g in range(C // 16):
                dv16 = dv[pl.ds(g * 16, 16)]
                d = plsc.load_gather(denv, [dv16])
                av[pl.ds(g * 16, 16)] = wv[pl.ds(g * 16, 16)] / (d + 1e-16)
            pltpu.sync_copy(av, a_out.at[pl.ds(base, C)])
            return 0

        lax.fori_loop(0, NCH, chunk, 0)

    return k(w2, dst, den2)


def kernel(x, edge_index, sentence, W1, att_src1, att_dst1, bias1,
           W2, att_src2, att_dst2, bias2, W_res1, W_res2, W_act, b_act):
    ei = edge_index.astype(jnp.int32)
    src, dst = ei[0], ei[1]

    xp1, xr1, as1, ad1 = _prep1(x, W1, W_res1, att_src1, att_dst1)

    asp = as1.T.reshape(H1 * N)
    adp = ad1.T.reshape(H1 * N)
    xp_tab = xp1.reshape(N * H1, HC)
    featp, wp = _sc_agg1(xp_tab, src, dst, asp, adp)
    feat = featp[0] + featp[1]
    agg1 = feat.transpose(1, 0, 2).reshape(N, H1 * HC)
    wsum = wp[0] + wp[1]
    den1 = wsum[:, :, 0].T

    xp2, hr2 = _mid(agg1, den1, xp1, as1, ad1, xr1, bias1, W2, W_res2)
    as2 = (xp2 * att_src2[0, 0][None, :]).sum(-1)
    ad2 = (xp2 * att_dst2[0, 0][None, :]).sum(-1)

    featp2, wp2, w2 = _sc_agg2(xp2, src, dst, as2, ad2)
    agg2 = featp2[0] + featp2[1]
    den2p = wp2[0, :, 0] + wp2[1, :, 0]

    es2 = as2 + ad2
    ws2 = jnp.exp(jnp.maximum(es2, 0.2 * es2))
    den2 = den2p + ws2

    h_out, logits_col = _fin(agg2, den2p, xp2, es2, hr2, bias2, W_act)
    logits = logits_col.reshape(1, -1) + b_act
    action = jnp.argmax(logits, axis=1)

    alpha_e = _sc_alpha(w2, dst, den2)
    alpha_s = ws2 / (den2 + 1e-16)
    alpha2 = jnp.concatenate([alpha_e, alpha_s])[:, None]
    return action, h_out, alpha2, logits
